# Initial kernel scaffold; baseline (speedup 1.0000x reference)
import jax
import jax.numpy as jnp
from jax.experimental import pallas as pl


def kernel(x, router_kernel, gate_proj, up_proj, down_proj):
    raise NotImplementedError("write your pallas kernel here")


# speedup vs baseline: 30.6010x; 30.6010x over previous
import functools

import jax
import jax.numpy as jnp
from jax import lax
from jax.experimental import pallas as pl
from jax.experimental.pallas import tpu as pltpu
from jax.experimental.pallas import tpu_sc as plsc

T, D, F, E, K = 2048, 1024, 768, 64, 2
N = T * K
TILE = 64
P = 8192
P_TILES = P // TILE
NW = 32
CHUNK = 32
LANES = 16


def _router_body(x_ref, rk_ref, i1_ref, i2_ref, w1_ref, w2_ref):
    s = jnp.dot(x_ref[...], rk_ref[...], preferred_element_type=jnp.float32)
    iota = lax.broadcasted_iota(jnp.int32, s.shape, 1)
    m1 = jnp.max(s, axis=1, keepdims=True)
    i1 = jnp.min(jnp.where(s == m1, iota, E), axis=1, keepdims=True)
    s2 = jnp.where(iota == i1, -jnp.inf, s)
    m2 = jnp.max(s2, axis=1, keepdims=True)
    i2 = jnp.min(jnp.where(s2 == m2, iota, E), axis=1, keepdims=True)
    w1 = 1.0 / (1.0 + jnp.exp(m2 - m1))
    i1_ref[...] = i1
    i2_ref[...] = i2
    w1_ref[...] = w1
    w2_ref[...] = 1.0 - w1


def _router(xf, rk):
    i1, i2, w1, w2 = pl.pallas_call(
        _router_body,
        out_shape=[
            jax.ShapeDtypeStruct((T, 1), jnp.int32),
            jax.ShapeDtypeStruct((T, 1), jnp.int32),
            jax.ShapeDtypeStruct((T, 1), jnp.float32),
            jax.ShapeDtypeStruct((T, 1), jnp.float32),
        ],
    )(xf, rk)
    return i1[:, 0], i2[:, 0], w1[:, 0], w2[:, 0]


def _build_metadata(i1, i2, w1, w2):
    e_pairs = jnp.stack([i1, i2], axis=1).reshape(-1)
    w_pairs = jnp.stack([w1, w2], axis=1).reshape(-1)
    oh = (e_pairs[:, None] == jnp.arange(E)[None, :]).astype(jnp.int32)
    csum = jnp.cumsum(oh, axis=0)
    rank = jnp.sum(csum * oh, axis=1) - 1
    counts = csum[-1]
    padded = ((counts + TILE - 1) // TILE) * TILE
    starts = jnp.concatenate([jnp.zeros((1,), jnp.int32),
                              jnp.cumsum(padded)[:-1].astype(jnp.int32)])
    slot = (starts[e_pairs] + rank).astype(jnp.int32)
    tile_cum = jnp.cumsum(padded // TILE)
    tile_expert = jnp.minimum(
        jnp.searchsorted(tile_cum, jnp.arange(P_TILES), side="right"),
        E - 1).astype(jnp.int32)
    tok_of_pair = (jnp.arange(N, dtype=jnp.int32) // K)
    sorted_tok = jnp.zeros((P,), jnp.int32).at[slot].set(tok_of_pair)
    sorted_w = jnp.zeros((P,), jnp.float32).at[slot].set(w_pairs)
    sp = slot.reshape(T, K)
    return sorted_tok, sorted_w, tile_expert, sp[:, 0], sp[:, 1]


def _dispatch(xf, sorted_tok):
    mesh = plsc.VectorSubcoreMesh(core_axis_name="c", subcore_axis_name="s")
    rows_w = P // NW
    nch = rows_w // CHUNK

    @functools.partial(
        pl.kernel,
        out_type=jax.ShapeDtypeStruct((P, D), jnp.float32),
        mesh=mesh,
        scratch_types=[
            pltpu.VMEM((CHUNK,), jnp.int32),
            pltpu.VMEM((CHUNK, D), jnp.float32),
            pltpu.SemaphoreType.DMA,
        ],
    )
    def k(tok_hbm, x_hbm, xs_hbm, idx_v, rows_v, sem):
        wid = lax.axis_index("s") * 2 + lax.axis_index("c")
        base = wid * rows_w

        def body(c, carry):
            b = base + c * CHUNK
            pltpu.sync_copy(tok_hbm.at[pl.ds(b, CHUNK)], idx_v)
            pltpu.async_copy(x_hbm.at[idx_v], rows_v, sem).wait()
            pltpu.sync_copy(rows_v, xs_hbm.at[pl.ds(b, CHUNK)])
            return carry

        lax.fori_loop(0, nch, body, 0)

    return k(sorted_tok, xf)


def _ffn_body(te_ref, xs_ref, g_ref, u_ref, d_ref, sw_ref, ys_ref):
    xt = xs_ref[...]
    g = jnp.dot(xt, g_ref[0], preferred_element_type=jnp.float32)
    u = jnp.dot(xt, u_ref[0], preferred_element_type=jnp.float32)
    h = g * jax.nn.sigmoid(g) * u
    y = jnp.dot(h, d_ref[0], preferred_element_type=jnp.float32)
    ys_ref[...] = y * sw_ref[0, 0, :][:, None]


def _ffn(xs, gate_proj, up_proj, down_proj, sorted_w, tile_expert):
    sw3 = sorted_w.reshape(P_TILES, 1, TILE)
    grid_spec = pltpu.PrefetchScalarGridSpec(
        num_scalar_prefetch=1,
        grid=(P_TILES,),
        in_specs=[
            pl.BlockSpec((TILE, D), lambda i, te: (i, 0)),
            pl.BlockSpec((1, D, F), lambda i, te: (te[i], 0, 0)),
            pl.BlockSpec((1, D, F), lambda i, te: (te[i], 0, 0)),
            pl.BlockSpec((1, F, D), lambda i, te: (te[i], 0, 0)),
            pl.BlockSpec((1, 1, TILE), lambda i, te: (i, 0, 0)),
        ],
        out_specs=pl.BlockSpec((TILE, D), lambda i, te: (i, 0)),
    )
    return pl.pallas_call(
        _ffn_body,
        grid_spec=grid_spec,
        out_shape=jax.ShapeDtypeStruct((P, D), jnp.float32),
    )(tile_expert, xs, gate_proj, up_proj, down_proj, sw3)


def _combine(ys, s1, s2):
    mesh = plsc.VectorSubcoreMesh(core_axis_name="c", subcore_axis_name="s")
    tok_w = T // NW
    nch = tok_w // CHUNK
    nvec = CHUNK * (D // LANES)

    @functools.partial(
        pl.kernel,
        out_type=jax.ShapeDtypeStruct((T, D), jnp.float32),
        mesh=mesh,
        scratch_types=[
            pltpu.VMEM((CHUNK,), jnp.int32),
            pltpu.VMEM((CHUNK,), jnp.int32),
            pltpu.VMEM((CHUNK, D), jnp.float32),
            pltpu.VMEM((CHUNK, D), jnp.float32),
            pltpu.SemaphoreType.DMA,
        ],
    )
    def k(s1_hbm, s2_hbm, ys_hbm, out_hbm, i1_v, i2_v, y1_v, y2_v, sem):
        wid = lax.axis_index("s") * 2 + lax.axis_index("c")
        base = wid * tok_w

        def chunk_body(c, carry):
            b = base + c * CHUNK
            pltpu.sync_copy(s1_hbm.at[pl.ds(b, CHUNK)], i1_v)
            pltpu.sync_copy(s2_hbm.at[pl.ds(b, CHUNK)], i2_v)
            pltpu.async_copy(ys_hbm.at[i1_v], y1_v, sem).wait()
            pltpu.async_copy(ys_hbm.at[i2_v], y2_v, sem).wait()

            def add_body(i, carry2):
                r = i // (D // LANES)
                col = (i % (D // LANES)) * LANES
                a = y1_v[r, pl.ds(col, LANES)]
                bvec = y2_v[r, pl.ds(col, LANES)]
                y1_v[r, pl.ds(col, LANES)] = a + bvec
                return carry2

            lax.fori_loop(0, nvec, add_body, 0)
            pltpu.sync_copy(y1_v, out_hbm.at[pl.ds(b, CHUNK)])
            return carry

        lax.fori_loop(0, nch, chunk_body, 0)

    return k(s1, s2, ys)


def kernel(x, router_kernel, gate_proj, up_proj, down_proj):
    b, t, d = x.shape
    xf = x.reshape(t, d)
    i1, i2, w1, w2 = _router(xf, router_kernel)
    sorted_tok, sorted_w, tile_expert, s1, s2 = _build_metadata(i1, i2, w1, w2)
    xs = _dispatch(xf, sorted_tok)
    ys = _ffn(xs, gate_proj, up_proj, down_proj, sorted_w, tile_expert)
    out = _combine(ys, s1, s2)
    return out.reshape(b, t, d)
